# Initial kernel scaffold; baseline (speedup 1.0000x reference)
import jax
import jax.numpy as jnp
from jax.experimental import pallas as pl


def kernel(x, edge_attr, params, edge_index, batch):
    raise NotImplementedError("write your pallas kernel here")


# speedup vs baseline: 13.4037x; 13.4037x over previous
import functools

import jax
import jax.numpy as jnp
from jax import lax
from jax.experimental import pallas as pl
from jax.experimental.pallas import tpu as pltpu
from jax.experimental.pallas import tpu_sc as plsc

N = 10000
NP = 10240
E = 160000
HID = 256
NH = 8
C = 32
G = 16
L = 8
NF = 10
EF = 4
XC = 288
AEC = 16
F32 = jnp.float32


def _ln(x, g, b):
    m = x.mean(-1, keepdims=True)
    v = ((x - m) ** 2).mean(-1, keepdims=True)
    return (x - m) / jnp.sqrt(v + 1e-5) * g + b


def _embed_body(x_ref, w1_ref, b1_ref, g_ref, bb_ref, w2_ref, b2_ref, o_ref):
    h = jnp.dot(x_ref[...], w1_ref[...], preferred_element_type=F32) + b1_ref[0]
    h = _ln(h, g_ref[0], bb_ref[0])
    h = jax.nn.gelu(h)
    o_ref[...] = jnp.dot(h, w2_ref[...], preferred_element_type=F32) + b2_ref[0]


def _mm_body(h_ref, w_ref, o_ref):
    o_ref[...] = jnp.dot(h_ref[...], w_ref[...], preferred_element_type=F32)


def _combine_body(acc_ref, xa_ref, cnt_ref, hin_ref, b_ref, g_ref, bb_ref, o_ref):
    acc = acc_ref[...]
    xa = xa_ref[...]
    xh = xa[:, :HID]
    a_d = xa[:, HID:HID + NH]
    a_s = xa[:, HID + 16:HID + 16 + NH]
    den8 = acc[:, HID:HID + NH]
    aes8 = acc[:, HID + 2 * NH:HID + 3 * NH]
    cnt = cnt_ref[...]
    ael = aes8 / jnp.maximum(cnt, 1.0)
    z = a_s + a_d + ael
    z = jnp.where(z >= 0, z, 0.2 * z)
    exs = jnp.exp(z)
    heads = lax.broadcasted_iota(jnp.int32, (NH, HID), 1) // C
    rows = lax.broadcasted_iota(jnp.int32, (NH, HID), 0)
    expm = jnp.where(heads == rows, 1.0, 0.0)
    num = acc[:, :HID] + jnp.dot(exs, expm, preferred_element_type=F32) * xh
    den = jnp.dot(den8 + exs, expm, preferred_element_type=F32)
    h2 = num / den + b_ref[0]
    h2 = jax.nn.gelu(_ln(h2, g_ref[0], bb_ref[0]))
    o_ref[...] = h2 + hin_ref[...]


def _pool_body(h_ref, b_ref, sum_ref, max_ref, cnt_ref):
    i = pl.program_id(0)

    @pl.when(i == 0)
    def _():
        sum_ref[...] = jnp.zeros_like(sum_ref)
        max_ref[...] = jnp.full_like(max_ref, -jnp.inf)
        cnt_ref[...] = jnp.zeros_like(cnt_ref)

    h = h_ref[...]
    bid = b_ref[...]
    bn = h.shape[0]
    gids = lax.broadcasted_iota(jnp.int32, (G, bn), 0)
    onehot = (gids == bid[:, 0][None, :]).astype(F32)
    sum_ref[...] += jnp.dot(onehot, h, preferred_element_type=F32)
    cnt_ref[...] += jnp.dot(onehot, jnp.ones((bn, NH), F32),
                            preferred_element_type=F32)
    rows = [jnp.max(jnp.where(bid == g, h, -jnp.inf), axis=0, keepdims=True)
            for g in range(G)]
    max_ref[...] = jnp.maximum(max_ref[...], jnp.concatenate(rows, axis=0))


def _heads_body(*refs):
    sum_ref, max_ref, cnt_ref = refs[0], refs[1], refs[2]
    wrefs = refs[3:-2]
    o1_ref, o2_ref = refs[-2], refs[-1]
    s = sum_ref[...]
    mx = max_ref[...]
    mx = jnp.where(mx > -1e37, mx, 0.0)
    c = jnp.maximum(cnt_ref[:, :1], 1.0)
    xg = jnp.concatenate([s / c, s, mx], axis=1)

    def head(ws):
        (w1, b1, g1, bb1, w2, b2, g2, bb2, w3, b3, g3, bb3, w4, b4) = ws
        h = xg @ w1[...] + b1[0]
        h = jax.nn.gelu(_ln(h, g1[0], bb1[0]))
        h = h @ w2[...] + b2[0]
        h = jax.nn.gelu(_ln(h, g2[0], bb2[0]))
        h = h @ w3[...] + b3[0]
        h = jax.nn.gelu(_ln(h, g3[0], bb3[0]))
        return jax.nn.sigmoid(h @ w4[...] + b4[0])

    o1_ref[...] = head(wrefs[:14])
    o2_ref[...] = head(wrefs[14:])


def _full(shape):
    return pl.BlockSpec(shape, lambda i: (0,) * len(shape))


def _embed_call(xp, w1, b1, g, bb, w2, b2, bn, out_cols):
    rows = xp.shape[0]
    k = xp.shape[1]
    grid = rows // bn
    return pl.pallas_call(
        _embed_body,
        grid=(grid,),
        in_specs=[
            pl.BlockSpec((bn, k), lambda i: (i, 0)),
            _full((k, HID)), _full((1, HID)), _full((1, HID)), _full((1, HID)),
            _full((HID, out_cols)), _full((1, out_cols)),
        ],
        out_specs=pl.BlockSpec((bn, out_cols), lambda i: (i, 0)),
        out_shape=jax.ShapeDtypeStruct((rows, out_cols), F32),
    )(xp, w1, b1[None], g[None], bb[None], w2, b2[None])


def _mm_call(h, w, bn):
    rows, k = h.shape
    cols = w.shape[1]
    return pl.pallas_call(
        _mm_body,
        grid=(rows // bn,),
        in_specs=[pl.BlockSpec((bn, k), lambda i: (i, 0)), _full((k, cols))],
        out_specs=pl.BlockSpec((bn, cols), lambda i: (i, 0)),
        out_shape=jax.ShapeDtypeStruct((rows, cols), F32),
    )(h, w)


def _combine_call(acc, xa, cntf, hin, b, g, bb, bn):
    grid = NP // bn
    return pl.pallas_call(
        _combine_body,
        grid=(grid,),
        in_specs=[
            pl.BlockSpec((bn, XC), lambda i: (i, 0)),
            pl.BlockSpec((bn, XC), lambda i: (i, 0)),
            pl.BlockSpec((bn, NH), lambda i: (i, 0)),
            pl.BlockSpec((bn, HID), lambda i: (i, 0)),
            _full((1, HID)), _full((1, HID)), _full((1, HID)),
        ],
        out_specs=pl.BlockSpec((bn, HID), lambda i: (i, 0)),
        out_shape=jax.ShapeDtypeStruct((NP, HID), F32),
    )(acc, xa, cntf, hin, b[None], g[None], bb[None])


def _pool_call(h, bid, bn):
    grid = NP // bn
    return pl.pallas_call(
        _pool_body,
        grid=(grid,),
        in_specs=[
            pl.BlockSpec((bn, HID), lambda i: (i, 0)),
            pl.BlockSpec((bn, 1), lambda i: (i, 0)),
        ],
        out_specs=[
            pl.BlockSpec((G, HID), lambda i: (0, 0)),
            pl.BlockSpec((G, HID), lambda i: (0, 0)),
            pl.BlockSpec((G, NH), lambda i: (0, 0)),
        ],
        out_shape=[
            jax.ShapeDtypeStruct((G, HID), F32),
            jax.ShapeDtypeStruct((G, HID), F32),
            jax.ShapeDtypeStruct((G, NH), F32),
        ],
    )(h, bid)


def _heads_call(sums, maxs, cnts, hp1, hp2):
    def wlist(p):
        out = []
        for wk, bk, gk, bbk in (('W1', 'b1', 'ln1_g', 'ln1_b'),
                                ('W2', 'b2', 'ln2_g', 'ln2_b'),
                                ('W3', 'b3', 'ln3_g', 'ln3_b')):
            out += [p[wk], p[bk][None], p[gk][None], p[bbk][None]]
        out += [p['W4'], p['b4'][None]]
        return out

    ws = wlist(hp1) + wlist(hp2)
    specs = [_full(w.shape) for w in ws]
    return pl.pallas_call(
        _heads_body,
        grid=(1,),
        in_specs=[_full((G, HID)), _full((G, HID)), _full((G, NH))] + specs,
        out_specs=[_full((G, 3)), _full((G, 3))],
        out_shape=[jax.ShapeDtypeStruct((G, 3), F32),
                   jax.ShapeDtypeStruct((G, 3), F32)],
    )(sums, maxs, cnts, *ws)


NPW = 320
RPPAD = NPW + 16
SRCPAD = 64


def _sc_wid():
    return lax.axis_index("s") * 2 + lax.axis_index("c")


def _edge_sc_kernel(xa_hbm, ae_hbm, src_hbm, rp_hbm, out_hbm,
                    rp_v, ownad_v, src_v, idx_v, rows_v, aer_v, out_v, sem):
    wid = _sc_wid()
    n0 = pl.multiple_of(wid * NPW, NPW)

    iota = lax.iota(jnp.int32, 16)

    pltpu.sync_copy(rp_hbm.at[pl.ds(n0, RPPAD)], rp_v)
    pltpu.sync_copy(xa_hbm.at[pl.ds(n0, NPW), pl.ds(HID, 16)], ownad_v)

    def _extract(ref, off):
        v = ref[pl.ds(off, 16)]
        return jnp.sum(jnp.where(iota == 0, v, 0))

    def body_node(m, p_prev):
        p0 = p_prev
        p1 = _extract(rp_v, m + 1)
        adv = ownad_v[m, :]

        nch = lax.div(p1 - p0 + 15, 16)
        acc_init = tuple(jnp.zeros((16,), F32) for _ in range(18))

        def body_chunk(ck, accs):
            p = p0 + ck * 16
            palign = pl.multiple_of(p - lax.rem(p, 8), 8)
            pltpu.sync_copy(src_hbm.at[pl.ds(palign, 32)], src_v)
            idx_v[...] = src_v[pl.ds(p - palign, 16)]
            cp = pltpu.make_async_copy(xa_hbm.at[idx_v], rows_v, sem)
            cp.start()
            pltpu.sync_copy(ae_hbm.at[pl.ds(p, 16), pl.ds(0, AEC)], aer_v)
            cp.wait()
            accs = list(accs)
            for j in range(16):
                asv = rows_v[j, pl.ds(HID + 16, 16)]
                aev = aer_v[j, :]
                z = asv + adv + aev
                z = jnp.where(z >= 0.0, z, 0.2 * z)
                wj = jnp.where(p + j < p1, 1.0, 0.0)
                ex = jnp.exp(z) * wj
                accs[16] = accs[16] + jnp.where(iota < 8, ex, 0.0)
                accs[17] = accs[17] + aev * wj
                for h in range(NH):
                    exh = jnp.sum(jnp.where(iota == h, ex, 0.0))
                    accs[2 * h] = accs[2 * h] + exh * rows_v[j, pl.ds(h * 32, 16)]
                    accs[2 * h + 1] = (accs[2 * h + 1]
                                       + exh * rows_v[j, pl.ds(h * 32 + 16, 16)])
            return tuple(accs)

        accs = lax.fori_loop(0, nch, body_chunk, acc_init)
        for k in range(16):
            out_v[m, pl.ds(k * 16, 16)] = accs[k]
        out_v[m, pl.ds(256, 16)] = accs[16]
        out_v[m, pl.ds(272, 16)] = accs[17]
        return p1

    e0 = _extract(rp_v, 0)
    lax.fori_loop(0, NPW, body_node, e0)
    pltpu.sync_copy(out_v, out_hbm.at[pl.ds(n0, NPW)])


def _edge_sc_call(xa, ae_l, src_pad, rp_pad):
    mesh = plsc.VectorSubcoreMesh(core_axis_name="c", subcore_axis_name="s",
                                  num_cores=2, num_subcores=16)
    kfn = pl.kernel(
        _edge_sc_kernel,
        mesh=mesh,
        compiler_params=pltpu.CompilerParams(use_tc_tiling_on_sc=False,
                                             needs_layout_passes=False),
        out_type=jax.ShapeDtypeStruct((NP, XC), F32),
        scratch_types=[
            pltpu.VMEM((RPPAD,), jnp.int32),
            pltpu.VMEM((NPW, 16), F32),
            pltpu.VMEM((32,), jnp.int32),
            pltpu.VMEM((16,), jnp.int32),
            pltpu.VMEM((16, XC), F32),
            pltpu.VMEM((16, AEC), F32),
            pltpu.VMEM((NPW, XC), F32),
            pltpu.SemaphoreType.DMA,
        ],
    )
    return kfn(xa, ae_l, src_pad, rp_pad)


def _fold_params(params):
    f = {}
    pe = params['edge_embed']
    ve = []
    for l in range(L):
        lp = params['layers'][l]
        ve.append((lp['W_e'].reshape(HID, NH, C) * lp['att_e'][None]).sum(-1))
    w2v = jnp.zeros((HID, L * AEC), F32)
    b2v = jnp.zeros((L * AEC,), F32)
    for l in range(L):
        w2v = w2v.at[:, l * AEC:l * AEC + NH].set(pe['W2'] @ ve[l])
        b2v = b2v.at[l * AEC:l * AEC + NH].set(pe['b2'] @ ve[l])
    f['w2v'] = w2v
    f['b2v'] = b2v
    wx = []
    for l in range(L):
        lp = params['layers'][l]
        asd = jnp.concatenate([
            (lp['W'].reshape(HID, NH, C) * lp['att_src'][None]).sum(-1),
            (lp['W'].reshape(HID, NH, C) * lp['att_dst'][None]).sum(-1)], axis=1)
        z8 = jnp.zeros((HID, NH), F32)
        wx.append(jnp.concatenate(
            [lp['W'], asd[:, NH:], z8, asd[:, :NH], z8], axis=1))
    f['wx'] = wx
    return f


@jax.jit
def kernel(x, edge_attr, params, edge_index, batch):
    src = edge_index[0]
    dst = edge_index[1]

    perm = jnp.argsort(dst)
    dst_s = dst[perm]
    src_s = src[perm]
    rp = jnp.searchsorted(dst_s, jnp.arange(N + 1, dtype=jnp.int32),
                          side='left').astype(jnp.int32)
    rp_pad = jnp.concatenate([rp, jnp.full((NP - N + RPPAD,), E, jnp.int32)])
    cntf = (jnp.maximum(jnp.diff(rp_pad[:NP + 1]).astype(F32), 0.0)[:, None]
            * jnp.ones((1, NH), F32))
    src_pad = jnp.concatenate([src_s, jnp.zeros((SRCPAD + 8,), jnp.int32)])

    fold = _fold_params(params)

    xp = jnp.zeros((NP, 16), F32).at[:N, :NF].set(x)
    pn = params['node_embed']
    w1p = jnp.concatenate([pn['W1'], jnp.zeros((16 - NF, HID), F32)], axis=0)
    h = _embed_call(xp, w1p, pn['b1'], pn['ln_g'], pn['ln_b'],
                    pn['W2'], pn['b2'], 2048, HID)

    ea_s = edge_attr[perm]
    eap = jnp.concatenate([ea_s, jnp.zeros((E, 8 - EF), F32)], axis=1)
    pe = params['edge_embed']
    w1e = jnp.concatenate([pe['W1'], jnp.zeros((8 - EF, HID), F32)], axis=0)
    ae_all = _embed_call(eap, w1e, pe['b1'], pe['ln_g'], pe['ln_b'],
                         fold['w2v'], fold['b2v'], 2000, L * AEC)
    ae_lay = ae_all.reshape(E, L, AEC).transpose(1, 0, 2)
    ae_lay = jnp.concatenate(
        [ae_lay, jnp.zeros((L, SRCPAD + 16, AEC), F32)], axis=1)

    for l in range(L):
        lp = params['layers'][l]
        xa = _mm_call(h, fold['wx'][l], 2048)
        acc = _edge_sc_call(xa, ae_lay[l], src_pad, rp_pad)
        hin = h if l > 0 else jnp.zeros((NP, HID), F32)
        h = _combine_call(acc, xa, cntf, hin,
                          lp['b'], lp['ln_g'], lp['ln_b'], 2048)

    batch_pad = jnp.concatenate([batch, jnp.full((NP - N,), G, jnp.int32)])
    sums, maxs, cnts = _pool_call(h, batch_pad[:, None], 640)
    o1, o2 = _heads_call(sums, maxs, cnts,
                         params['param_mlp'], params['metrics_mlp'])
    return o1, o2
